# baseline (speedup 1.0000x reference)
import functools

import jax
import jax.numpy as jnp
from jax import lax
from jax.experimental import pallas as pl
from jax.experimental.pallas import tpu as pltpu
from jax.experimental.pallas import tpu_sc as plsc

L = 2
E = 8
K = 2
D = 1024
F = 704
FS = 1408
T = 2048

TBG = 256
NB = T * K // TBG + E - 1
NSLOT = NB * TBG
STB = 256
CH = 256

NW = 32
BPW = T // NW


def _sc_mesh():
    return plsc.VectorSubcoreMesh(core_axis_name="c", subcore_axis_name="s")


def _embed_gather(embed_table, input_ids):
    @functools.partial(
        pl.kernel,
        mesh=_sc_mesh(),
        out_type=jax.ShapeDtypeStruct((T, D), jnp.float32),
        scratch_types=[
            pltpu.VMEM((BPW,), jnp.int32),
            pltpu.VMEM((BPW, D), jnp.float32),
            pltpu.SemaphoreType.DMA,
        ],
    )
    def k(table_hbm, idx_hbm, out_hbm, idx_v, rows_v, sem):
        wid = lax.axis_index("s") * 2 + lax.axis_index("c")
        base = wid * BPW
        pltpu.sync_copy(idx_hbm.at[pl.ds(base, BPW)], idx_v)
        pltpu.async_copy(table_hbm.at[idx_v], rows_v, sem).wait()
        pltpu.sync_copy(rows_v, out_hbm.at[pl.ds(base, BPW)])

    return k(embed_table, input_ids)


def _dispatch(h, pos0, pos1):
    @functools.partial(
        pl.kernel,
        mesh=_sc_mesh(),
        out_type=jax.ShapeDtypeStruct((NSLOT, D), jnp.float32),
        scratch_types=[
            pltpu.VMEM((BPW,), jnp.int32),
            pltpu.VMEM((BPW,), jnp.int32),
            pltpu.VMEM((BPW, D), jnp.float32),
            pltpu.SemaphoreType.DMA,
        ],
    )
    def k(h_hbm, p0_hbm, p1_hbm, xs_hbm, i0_v, i1_v, rows_v, sem):
        wid = lax.axis_index("s") * 2 + lax.axis_index("c")
        base = wid * BPW
        pltpu.sync_copy(p0_hbm.at[pl.ds(base, BPW)], i0_v)
        pltpu.sync_copy(p1_hbm.at[pl.ds(base, BPW)], i1_v)
        pltpu.sync_copy(h_hbm.at[pl.ds(base, BPW)], rows_v)
        pltpu.async_copy(rows_v, xs_hbm.at[i0_v], sem).wait()
        pltpu.async_copy(rows_v, xs_hbm.at[i1_v], sem).wait()

    return k(h, pos0, pos1)


def _unsort(y_sorted, pos0, pos1):
    @functools.partial(
        pl.kernel,
        mesh=_sc_mesh(),
        out_type=(
            jax.ShapeDtypeStruct((T, D), jnp.float32),
            jax.ShapeDtypeStruct((T, D), jnp.float32),
        ),
        scratch_types=[
            pltpu.VMEM((BPW,), jnp.int32),
            pltpu.VMEM((BPW, D), jnp.float32),
            pltpu.SemaphoreType.DMA,
        ],
    )
    def k(ys_hbm, p0_hbm, p1_hbm, y0_hbm, y1_hbm, i_v, rows_v, sem):
        wid = lax.axis_index("s") * 2 + lax.axis_index("c")
        base = wid * BPW
        pltpu.sync_copy(p0_hbm.at[pl.ds(base, BPW)], i_v)
        pltpu.async_copy(ys_hbm.at[i_v], rows_v, sem).wait()
        pltpu.sync_copy(rows_v, y0_hbm.at[pl.ds(base, BPW)])
        pltpu.sync_copy(p1_hbm.at[pl.ds(base, BPW)], i_v)
        pltpu.async_copy(ys_hbm.at[i_v], rows_v, sem).wait()
        pltpu.sync_copy(rows_v, y1_hbm.at[pl.ds(base, BPW)])

    return k(y_sorted, pos0, pos1)


def _router_body(h_ref, rw_ref, pos0_ref, pos1_ref, w01_ref, spinfo_ref):
    h = h_ref[...]
    logits = jnp.dot(h, rw_ref[0], preferred_element_type=jnp.float32)

    m = jnp.max(logits, axis=-1, keepdims=True)
    ex = jnp.exp(logits - m)
    probs = ex / jnp.sum(ex, axis=-1, keepdims=True)
    lane = lax.broadcasted_iota(jnp.int32, probs.shape, 1)
    v1 = jnp.max(probs, axis=-1, keepdims=True)
    i1 = jnp.min(jnp.where(probs == v1, lane, E), axis=-1, keepdims=True)
    pm = jnp.where(lane == i1, -jnp.inf, probs)
    v2 = jnp.max(pm, axis=-1, keepdims=True)
    i2 = jnp.min(jnp.where(pm == v2, lane, E), axis=-1, keepdims=True)
    w01_ref[...] = jnp.concatenate([v1, v2], axis=1)

    cnt = jnp.where(lane == i1, 1.0, 0.0) + jnp.where(lane == i2, 1.0, 0.0)

    r = lax.broadcasted_iota(jnp.int32, (CH, CH), 0)
    c = lax.broadcasted_iota(jnp.int32, (CH, CH), 1)
    tri = jnp.where(r > c, 1.0, 0.0)
    run = jnp.zeros((1, E), jnp.float32)
    chunks = []
    for j in range(T // CH):
        cc = cnt[j * CH:(j + 1) * CH]
        chunks.append(jnp.dot(tri, cc, preferred_element_type=jnp.float32) + run)
        run = run + jnp.sum(cc, axis=0, keepdims=True)
    cprefix = jnp.concatenate(chunks, axis=0)
    n = run

    pn = jnp.ceil(n / TBG) * TBG
    r8 = lax.broadcasted_iota(jnp.int32, (E, E), 0)
    c8 = lax.broadcasted_iota(jnp.int32, (E, E), 1)
    ustrict = jnp.where(r8 < c8, 1.0, 0.0)
    off = jnp.dot(pn, ustrict, preferred_element_type=jnp.float32)
    cum = off + pn
    ptotal = jnp.sum(pn)

    def sel(arr, key):
        return jnp.sum(jnp.where(lane == key, arr, 0.0), axis=1, keepdims=True)

    pos0 = sel(off, i1) + sel(cprefix, i1)
    pos1 = sel(off, i2) + sel(cprefix, i2)
    pos0_ref[...] = pos0.astype(jnp.int32)
    pos1_ref[...] = pos1.astype(jnp.int32)

    del cum, ptotal
    spinfo = jnp.concatenate([off / TBG, pn / TBG], axis=1)
    spinfo_ref[...] = spinfo.astype(jnp.int32)


def _router(h, router_w, l):
    return pl.pallas_call(
        _router_body,
        grid=(1,),
        in_specs=[
            pl.BlockSpec((T, D), lambda i: (0, 0)),
            pl.BlockSpec((1, D, E), lambda i: (l, 0, 0)),
        ],
        out_specs=[
            pl.BlockSpec((T, 1), lambda i: (0, 0)),
            pl.BlockSpec((T, 1), lambda i: (0, 0)),
            pl.BlockSpec((T, K), lambda i: (0, 0)),
            pl.BlockSpec((1, 2 * E), lambda i: (0, 0)),
        ],
        out_shape=[
            jax.ShapeDtypeStruct((T, 1), jnp.int32),
            jax.ShapeDtypeStruct((T, 1), jnp.int32),
            jax.ShapeDtypeStruct((T, K), jnp.float32),
            jax.ShapeDtypeStruct((1, 2 * E), jnp.int32),
        ],
    )(h, router_w)


def _grouped_body(sp_ref, x_ref, wg_ref, wu_ref, wd_ref, y_ref):
    e = pl.program_id(0)
    j = pl.program_id(1)

    @pl.when(j < sp_ref[E + e])
    def _():
        nt = (((1,), (1,)), ((), ()))
        x = x_ref[...].astype(jnp.bfloat16)
        g = lax.dot_general(x, wg_ref[0, 0].astype(jnp.bfloat16), nt,
                            preferred_element_type=jnp.float32)
        u = lax.dot_general(x, wu_ref[0, 0].astype(jnp.bfloat16), nt,
                            preferred_element_type=jnp.float32)
        a = (jax.nn.silu(g) * u).astype(jnp.bfloat16)
        y_ref[...] = jnp.dot(a, wd_ref[0, 0].astype(jnp.bfloat16),
                             preferred_element_type=jnp.float32)


MB = T // TBG


def _grouped_ffn(spinfo, x_sorted, w_gate, w_up, w_down, l):
    def slot_ix(e, j, sp):
        return (jnp.maximum(jnp.minimum(sp[e] + j, sp[e] + sp[E + e] - 1), 0), 0)

    grid_spec = pltpu.PrefetchScalarGridSpec(
        num_scalar_prefetch=1,
        grid=(E, MB),
        in_specs=[
            pl.BlockSpec((TBG, D), slot_ix),
            pl.BlockSpec((1, 1, F, D), lambda e, j, sp: (l, e, 0, 0)),
            pl.BlockSpec((1, 1, F, D), lambda e, j, sp: (l, e, 0, 0)),
            pl.BlockSpec((1, 1, F, D), lambda e, j, sp: (l, e, 0, 0)),
        ],
        out_specs=pl.BlockSpec((TBG, D), slot_ix),
    )
    return pl.pallas_call(
        _grouped_body,
        grid_spec=grid_spec,
        out_shape=jax.ShapeDtypeStruct((NSLOT, D), jnp.float32),
        compiler_params=pltpu.CompilerParams(
            dimension_semantics=("arbitrary", "arbitrary")),
    )(spinfo, x_sorted, w_gate, w_up, w_down)


def _shared_body(h_ref, s1_ref, s3_ref, s2_ref, sg_ref, out_ref):
    h = h_ref[...]
    hb = h.astype(jnp.bfloat16)
    a = jax.nn.silu(jnp.dot(hb, s1_ref[0].astype(jnp.bfloat16),
                            preferred_element_type=jnp.float32))
    b = jnp.dot(hb, s3_ref[0].astype(jnp.bfloat16),
                preferred_element_type=jnp.float32)
    sh = jnp.dot((a * b).astype(jnp.bfloat16), s2_ref[0].astype(jnp.bfloat16),
                 preferred_element_type=jnp.float32)
    gate = jax.nn.sigmoid(jnp.dot(h, sg_ref[0],
                                  preferred_element_type=jnp.float32))
    out_ref[...] = gate * sh


def _shared(h, sh_w1, sh_w3, sh_w2, sh_gate, l):
    return pl.pallas_call(
        _shared_body,
        grid=(T // STB,),
        in_specs=[
            pl.BlockSpec((STB, D), lambda t: (t, 0)),
            pl.BlockSpec((1, D, FS), lambda t: (l, 0, 0)),
            pl.BlockSpec((1, D, FS), lambda t: (l, 0, 0)),
            pl.BlockSpec((1, FS, D), lambda t: (l, 0, 0)),
            pl.BlockSpec((1, D, 1), lambda t: (l, 0, 0)),
        ],
        out_specs=pl.BlockSpec((STB, D), lambda t: (t, 0)),
        out_shape=jax.ShapeDtypeStruct((T, D), jnp.float32),
        compiler_params=pltpu.CompilerParams(
            dimension_semantics=("parallel",)),
    )(h, sh_w1, sh_w3, sh_w2, sh_gate)


def _combine_body(y0_ref, y1_ref, w01_ref, sh_ref, out_ref):
    w01 = w01_ref[...]
    out_ref[...] = (w01[:, 0:1] * y0_ref[...] + w01[:, 1:2] * y1_ref[...]
                    + sh_ref[...])


def _combine(y0, y1, w01, sh):
    return pl.pallas_call(
        _combine_body,
        grid=(T // STB,),
        in_specs=[
            pl.BlockSpec((STB, D), lambda t: (t, 0)),
            pl.BlockSpec((STB, D), lambda t: (t, 0)),
            pl.BlockSpec((STB, K), lambda t: (t, 0)),
            pl.BlockSpec((STB, D), lambda t: (t, 0)),
        ],
        out_specs=pl.BlockSpec((STB, D), lambda t: (t, 0)),
        out_shape=jax.ShapeDtypeStruct((T, D), jnp.float32),
        compiler_params=pltpu.CompilerParams(
            dimension_semantics=("parallel",)),
    )(y0, y1, w01, sh)


@jax.jit
def kernel(input_ids, positions, embed_table, router_w, w_gate, w_up, w_down, sh_w1, sh_w3, sh_w2, sh_gate):
    del positions
    h = _embed_gather(embed_table, input_ids.astype(jnp.int32))
    for l in range(L):
        pos0, pos1, w01, spinfo = _router(h, router_w, l)
        p0 = jnp.reshape(pos0, (T,))
        p1 = jnp.reshape(pos1, (T,))
        x_sorted = _dispatch(h, p0, p1)
        sh = _shared(h, sh_w1, sh_w3, sh_w2, sh_gate, l)
        y_sorted = _grouped_ffn(jnp.reshape(spinfo, (2 * E,)), x_sorted,
                                jnp.swapaxes(w_gate, 2, 3),
                                jnp.swapaxes(w_up, 2, 3), w_down, l)
        y0, y1 = _unsort(y_sorted, p0, p1)
        h = _combine(y0, y1, w01, sh)
    return h


# speedup vs baseline: 1.1060x; 1.1060x over previous
import functools

import jax
import jax.numpy as jnp
from jax import lax
from jax.experimental import pallas as pl
from jax.experimental.pallas import tpu as pltpu
from jax.experimental.pallas import tpu_sc as plsc

L = 2
E = 8
K = 2
D = 1024
F = 704
FS = 1408
T = 2048

TBG = 512
NB = T * K // TBG + E - 1
NSLOT = NB * TBG
STB = 256
CH = 256

NW = 32
BPW = T // NW


def _sc_mesh():
    return plsc.VectorSubcoreMesh(core_axis_name="c", subcore_axis_name="s")


def _embed_gather(embed_table, input_ids):
    @functools.partial(
        pl.kernel,
        mesh=_sc_mesh(),
        out_type=jax.ShapeDtypeStruct((T, D), jnp.float32),
        scratch_types=[
            pltpu.VMEM((BPW,), jnp.int32),
            pltpu.VMEM((BPW, D), jnp.float32),
            pltpu.SemaphoreType.DMA,
        ],
    )
    def k(table_hbm, idx_hbm, out_hbm, idx_v, rows_v, sem):
        wid = lax.axis_index("s") * 2 + lax.axis_index("c")
        base = wid * BPW
        pltpu.sync_copy(idx_hbm.at[pl.ds(base, BPW)], idx_v)
        pltpu.async_copy(table_hbm.at[idx_v], rows_v, sem).wait()
        pltpu.sync_copy(rows_v, out_hbm.at[pl.ds(base, BPW)])

    return k(embed_table, input_ids)


def _dispatch(h, pos0, pos1):
    @functools.partial(
        pl.kernel,
        mesh=_sc_mesh(),
        out_type=jax.ShapeDtypeStruct((NSLOT, D), jnp.float32),
        scratch_types=[
            pltpu.VMEM((BPW,), jnp.int32),
            pltpu.VMEM((BPW,), jnp.int32),
            pltpu.VMEM((BPW, D), jnp.float32),
            pltpu.SemaphoreType.DMA,
        ],
    )
    def k(h_hbm, p0_hbm, p1_hbm, xs_hbm, i0_v, i1_v, rows_v, sem):
        wid = lax.axis_index("s") * 2 + lax.axis_index("c")
        base = wid * BPW
        pltpu.sync_copy(p0_hbm.at[pl.ds(base, BPW)], i0_v)
        pltpu.sync_copy(p1_hbm.at[pl.ds(base, BPW)], i1_v)
        pltpu.sync_copy(h_hbm.at[pl.ds(base, BPW)], rows_v)
        pltpu.async_copy(rows_v, xs_hbm.at[i0_v], sem).wait()
        pltpu.async_copy(rows_v, xs_hbm.at[i1_v], sem).wait()

    return k(h, pos0, pos1)


def _unsort(y_sorted, pos0, pos1):
    @functools.partial(
        pl.kernel,
        mesh=_sc_mesh(),
        out_type=(
            jax.ShapeDtypeStruct((T, D), jnp.float32),
            jax.ShapeDtypeStruct((T, D), jnp.float32),
        ),
        scratch_types=[
            pltpu.VMEM((BPW,), jnp.int32),
            pltpu.VMEM((BPW, D), jnp.float32),
            pltpu.SemaphoreType.DMA,
        ],
    )
    def k(ys_hbm, p0_hbm, p1_hbm, y0_hbm, y1_hbm, i_v, rows_v, sem):
        wid = lax.axis_index("s") * 2 + lax.axis_index("c")
        base = wid * BPW
        pltpu.sync_copy(p0_hbm.at[pl.ds(base, BPW)], i_v)
        pltpu.async_copy(ys_hbm.at[i_v], rows_v, sem).wait()
        pltpu.sync_copy(rows_v, y0_hbm.at[pl.ds(base, BPW)])
        pltpu.sync_copy(p1_hbm.at[pl.ds(base, BPW)], i_v)
        pltpu.async_copy(ys_hbm.at[i_v], rows_v, sem).wait()
        pltpu.sync_copy(rows_v, y1_hbm.at[pl.ds(base, BPW)])

    return k(y_sorted, pos0, pos1)


def _router_body(h_ref, rw_ref, pos0_ref, pos1_ref, w01_ref, spinfo_ref):
    h = h_ref[...]
    logits = jnp.dot(h, rw_ref[0], preferred_element_type=jnp.float32)

    m = jnp.max(logits, axis=-1, keepdims=True)
    ex = jnp.exp(logits - m)
    probs = ex / jnp.sum(ex, axis=-1, keepdims=True)
    lane = lax.broadcasted_iota(jnp.int32, probs.shape, 1)
    v1 = jnp.max(probs, axis=-1, keepdims=True)
    i1 = jnp.min(jnp.where(probs == v1, lane, E), axis=-1, keepdims=True)
    pm = jnp.where(lane == i1, -jnp.inf, probs)
    v2 = jnp.max(pm, axis=-1, keepdims=True)
    i2 = jnp.min(jnp.where(pm == v2, lane, E), axis=-1, keepdims=True)
    w01_ref[...] = jnp.concatenate([v1, v2], axis=1)

    cnt = jnp.where(lane == i1, 1.0, 0.0) + jnp.where(lane == i2, 1.0, 0.0)

    r = lax.broadcasted_iota(jnp.int32, (CH, CH), 0)
    c = lax.broadcasted_iota(jnp.int32, (CH, CH), 1)
    tri = jnp.where(r > c, 1.0, 0.0)
    run = jnp.zeros((1, E), jnp.float32)
    chunks = []
    for j in range(T // CH):
        cc = cnt[j * CH:(j + 1) * CH]
        chunks.append(jnp.dot(tri, cc, preferred_element_type=jnp.float32) + run)
        run = run + jnp.sum(cc, axis=0, keepdims=True)
    cprefix = jnp.concatenate(chunks, axis=0)
    n = run

    pn = jnp.ceil(n / TBG) * TBG
    r8 = lax.broadcasted_iota(jnp.int32, (E, E), 0)
    c8 = lax.broadcasted_iota(jnp.int32, (E, E), 1)
    ustrict = jnp.where(r8 < c8, 1.0, 0.0)
    off = jnp.dot(pn, ustrict, preferred_element_type=jnp.float32)
    cum = off + pn
    ptotal = jnp.sum(pn)

    def sel(arr, key):
        return jnp.sum(jnp.where(lane == key, arr, 0.0), axis=1, keepdims=True)

    pos0 = sel(off, i1) + sel(cprefix, i1)
    pos1 = sel(off, i2) + sel(cprefix, i2)
    pos0_ref[...] = pos0.astype(jnp.int32)
    pos1_ref[...] = pos1.astype(jnp.int32)

    bs = (lax.broadcasted_iota(jnp.int32, (1, NB), 1) * TBG).astype(jnp.float32)
    bs = jnp.minimum(bs, ptotal - 1.0)
    be = jnp.zeros((1, NB), jnp.float32)
    for e in range(E):
        be = be + jnp.where(bs >= cum[0, e], 1.0, 0.0)
    spinfo = jnp.concatenate(
        [be, jnp.full((1, 1), ptotal, jnp.float32)], axis=1)
    spinfo_ref[...] = spinfo.astype(jnp.int32)


def _router(h, router_w, l):
    return pl.pallas_call(
        _router_body,
        grid=(1,),
        in_specs=[
            pl.BlockSpec((T, D), lambda i: (0, 0)),
            pl.BlockSpec((1, D, E), lambda i: (l, 0, 0)),
        ],
        out_specs=[
            pl.BlockSpec((T, 1), lambda i: (0, 0)),
            pl.BlockSpec((T, 1), lambda i: (0, 0)),
            pl.BlockSpec((T, K), lambda i: (0, 0)),
            pl.BlockSpec((1, NB + 1), lambda i: (0, 0)),
        ],
        out_shape=[
            jax.ShapeDtypeStruct((T, 1), jnp.int32),
            jax.ShapeDtypeStruct((T, 1), jnp.int32),
            jax.ShapeDtypeStruct((T, K), jnp.float32),
            jax.ShapeDtypeStruct((1, NB + 1), jnp.int32),
        ],
    )(h, router_w)


def _grouped_body(sp_ref, x_ref, wg_ref, wu_ref, wd_ref, y_ref):
    b = pl.program_id(0)

    @pl.when(b * TBG < sp_ref[NB])
    def _():
        nt = (((1,), (1,)), ((), ()))
        x = x_ref[...].astype(jnp.bfloat16)
        g = lax.dot_general(x, wg_ref[0, 0].astype(jnp.bfloat16), nt,
                            preferred_element_type=jnp.float32)
        u = lax.dot_general(x, wu_ref[0, 0].astype(jnp.bfloat16), nt,
                            preferred_element_type=jnp.float32)
        a = (jax.nn.silu(g) * u).astype(jnp.bfloat16)
        y_ref[...] = jnp.dot(a, wd_ref[0, 0].astype(jnp.bfloat16),
                             preferred_element_type=jnp.float32)


def _grouped_ffn(spinfo, x_sorted, w_gate, w_up, w_down, l):
    grid_spec = pltpu.PrefetchScalarGridSpec(
        num_scalar_prefetch=1,
        grid=(NB,),
        in_specs=[
            pl.BlockSpec((TBG, D), lambda b, sp: (b, 0)),
            pl.BlockSpec((1, 1, F, D), lambda b, sp: (l, sp[b], 0, 0)),
            pl.BlockSpec((1, 1, F, D), lambda b, sp: (l, sp[b], 0, 0)),
            pl.BlockSpec((1, 1, F, D), lambda b, sp: (l, sp[b], 0, 0)),
        ],
        out_specs=pl.BlockSpec((TBG, D), lambda b, sp: (b, 0)),
    )
    return pl.pallas_call(
        _grouped_body,
        grid_spec=grid_spec,
        out_shape=jax.ShapeDtypeStruct((NSLOT, D), jnp.float32),
        compiler_params=pltpu.CompilerParams(
            dimension_semantics=("parallel",)),
    )(spinfo, x_sorted, w_gate, w_up, w_down)


def _shared_body(h_ref, s1_ref, s3_ref, s2_ref, sg_ref, out_ref):
    h = h_ref[...]
    hb = h.astype(jnp.bfloat16)
    a = jax.nn.silu(jnp.dot(hb, s1_ref[0].astype(jnp.bfloat16),
                            preferred_element_type=jnp.float32))
    b = jnp.dot(hb, s3_ref[0].astype(jnp.bfloat16),
                preferred_element_type=jnp.float32)
    sh = jnp.dot((a * b).astype(jnp.bfloat16), s2_ref[0].astype(jnp.bfloat16),
                 preferred_element_type=jnp.float32)
    gate = jax.nn.sigmoid(jnp.dot(h, sg_ref[0],
                                  preferred_element_type=jnp.float32))
    out_ref[...] = gate * sh


def _shared(h, sh_w1, sh_w3, sh_w2, sh_gate, l):
    return pl.pallas_call(
        _shared_body,
        grid=(T // STB,),
        in_specs=[
            pl.BlockSpec((STB, D), lambda t: (t, 0)),
            pl.BlockSpec((1, D, FS), lambda t: (l, 0, 0)),
            pl.BlockSpec((1, D, FS), lambda t: (l, 0, 0)),
            pl.BlockSpec((1, FS, D), lambda t: (l, 0, 0)),
            pl.BlockSpec((1, D, 1), lambda t: (l, 0, 0)),
        ],
        out_specs=pl.BlockSpec((STB, D), lambda t: (t, 0)),
        out_shape=jax.ShapeDtypeStruct((T, D), jnp.float32),
        compiler_params=pltpu.CompilerParams(
            dimension_semantics=("parallel",)),
    )(h, sh_w1, sh_w3, sh_w2, sh_gate)


def _combine_body(y0_ref, y1_ref, w01_ref, sh_ref, out_ref):
    w01 = w01_ref[...]
    out_ref[...] = (w01[:, 0:1] * y0_ref[...] + w01[:, 1:2] * y1_ref[...]
                    + sh_ref[...])


def _combine(y0, y1, w01, sh):
    return pl.pallas_call(
        _combine_body,
        grid=(T // STB,),
        in_specs=[
            pl.BlockSpec((STB, D), lambda t: (t, 0)),
            pl.BlockSpec((STB, D), lambda t: (t, 0)),
            pl.BlockSpec((STB, K), lambda t: (t, 0)),
            pl.BlockSpec((STB, D), lambda t: (t, 0)),
        ],
        out_specs=pl.BlockSpec((STB, D), lambda t: (t, 0)),
        out_shape=jax.ShapeDtypeStruct((T, D), jnp.float32),
        compiler_params=pltpu.CompilerParams(
            dimension_semantics=("parallel",)),
    )(y0, y1, w01, sh)


@jax.jit
def kernel(input_ids, positions, embed_table, router_w, w_gate, w_up, w_down, sh_w1, sh_w3, sh_w2, sh_gate):
    del positions
    h = _embed_gather(embed_table, input_ids.astype(jnp.int32))
    for l in range(L):
        pos0, pos1, w01, spinfo = _router(h, router_w, l)
        p0 = jnp.reshape(pos0, (T,))
        p1 = jnp.reshape(pos1, (T,))
        x_sorted = _dispatch(h, p0, p1)
        sh = _shared(h, sh_w1, sh_w3, sh_w2, sh_gate, l)
        y_sorted = _grouped_ffn(jnp.reshape(spinfo, (NB + 1,)), x_sorted,
                                jnp.swapaxes(w_gate, 2, 3),
                                jnp.swapaxes(w_up, 2, 3), w_down, l)
        y0, y1 = _unsort(y_sorted, p0, p1)
        h = _combine(y0, y1, w01, sh)
    return h
